# Initial kernel scaffold; baseline (speedup 1.0000x reference)
import jax
import jax.numpy as jnp
from jax.experimental import pallas as pl


def kernel(x, edge_index, W, a_src, a_dst, bias):
    raise NotImplementedError("write your pallas kernel here")


# speedup vs baseline: 26.6280x; 26.6280x over previous
import functools

import jax
import jax.numpy as jnp
from jax import lax
from jax.experimental import pallas as pl
from jax.experimental.pallas import tpu as pltpu
from jax.experimental.pallas import tpu_sc as plsc

N = 10000
E = 320000
D = 128

NUM_SC = 2
NUM_TILES = 16
NUM_WORKERS = NUM_SC * NUM_TILES
EPW = E // NUM_WORKERS
CHUNK = 80
NCHUNK = EPW // CHUNK
ZROWS = 640
ZSTRIDE = 632
BM = 2000


def _mm_body(x_ref, w_ref, a2_ref, h_ref, sat_ref):
    h = jnp.dot(x_ref[...], w_ref[...], preferred_element_type=jnp.float32)
    h_ref[...] = h
    sat_ref[...] = lax.dot_general(
        h, a2_ref[...], (((1,), (1,)), ((), ())),
        preferred_element_type=jnp.float32)


def _matmul(x, W, a2):
    return pl.pallas_call(
        _mm_body,
        grid=(N // BM,),
        in_specs=[
            pl.BlockSpec((BM, D), lambda i: (i, 0)),
            pl.BlockSpec((D, D), lambda i: (0, 0)),
            pl.BlockSpec((8, D), lambda i: (0, 0)),
        ],
        out_specs=[
            pl.BlockSpec((BM, D), lambda i: (i, 0)),
            pl.BlockSpec((BM, 8), lambda i: (i, 0)),
        ],
        out_shape=[
            jax.ShapeDtypeStruct((N, D), jnp.float32),
            jax.ShapeDtypeStruct((N, 8), jnp.float32),
        ],
    )(x, W, a2)


def _comb_body(u_ref, h_ref, v_ref, b_ref, o_ref):
    v = v_ref[...]
    ws = v[:, 0:1] + v[:, 1:2]
    ws = jnp.exp(jnp.where(ws >= 0.0, ws, 0.2 * ws))
    den = v[:, 2:3] + v[:, 3:4] + ws + 1e-16
    o_ref[...] = (u_ref[0] + u_ref[1] + ws * h_ref[...]) / den + b_ref[...]


def _combine(u, h, vecs, bias2d):
    return pl.pallas_call(
        _comb_body,
        grid=(N // BM,),
        in_specs=[
            pl.BlockSpec((2, BM, D), lambda i: (0, i, 0)),
            pl.BlockSpec((BM, D), lambda i: (i, 0)),
            pl.BlockSpec((BM, 4), lambda i: (i, 0)),
            pl.BlockSpec((1, D), lambda i: (0, 0)),
        ],
        out_specs=pl.BlockSpec((BM, D), lambda i: (i, 0)),
        out_shape=jax.ShapeDtypeStruct((N, D), jnp.float32),
    )(u, h, vecs, bias2d)


def _sc_edge_body(src_hbm, dst_hbm, as_hbm, ad_hbm, h_hbm,
                  u_out, den_out,
                  as_l, ad_l, src_b, dst_b, w_b, rows_b, zbuf, dzb, u_sh,
                  den_sh, sem):
    c = lax.axis_index("c")
    s = lax.axis_index("s")
    wid = s * NUM_SC + c

    pltpu.sync_copy(as_hbm, as_l)
    pltpu.sync_copy(ad_hbm, ad_l)

    z16 = jnp.zeros((16,), jnp.float32)
    for r in range(ZROWS // 8):
        for k in range(D // 16):
            zbuf[r, pl.ds(k * 16, 16)] = z16
    for i in range(ZROWS // 16):
        dzb[pl.ds(i * 16, 16)] = z16
    rstart = jnp.minimum(s * ZSTRIDE, N - ZROWS)
    for i in range(8):
        pltpu.sync_copy(zbuf, u_sh.at[pl.ds(rstart + i * (ZROWS // 8),
                                            ZROWS // 8)])
    pltpu.sync_copy(dzb, den_sh.at[pl.ds(rstart, ZROWS)])
    plsc.subcore_barrier()

    ebase = wid * EPW

    def chunk(j, carry):
        off = ebase + j * CHUNK
        pltpu.sync_copy(src_hbm.at[pl.ds(off, CHUNK)], src_b)
        pltpu.sync_copy(dst_hbm.at[pl.ds(off, CHUNK)], dst_b)
        for g in range(CHUNK // 16):
            si = src_b[pl.ds(g * 16, 16)]
            di = dst_b[pl.ds(g * 16, 16)]
            e = plsc.load_gather(as_l, [si]) + plsc.load_gather(ad_l, [di])
            e = jnp.where(e >= 0.0, e, e * 0.2)
            w_b[pl.ds(g * 16, 16)] = jnp.exp(e)
        pltpu.async_copy(h_hbm.at[src_b], rows_b, sem).wait()
        for g in range(CHUNK // 16):
            wv = w_b[pl.ds(g * 16, 16)]
            for l in range(16):
                ws = jnp.full((16,), wv[l], jnp.float32)
                r = g * 16 + l
                for k in range(D // 16):
                    rows_b[r, pl.ds(k * 16, 16)] = (
                        rows_b[r, pl.ds(k * 16, 16)] * ws)
        pltpu.sync_copy(w_b, den_sh.at[dst_b], add=True)
        pltpu.sync_copy(rows_b, u_sh.at[dst_b], add=True)
        return carry

    lax.fori_loop(0, NCHUNK, chunk, 0)
    plsc.subcore_barrier()

    for i in range(8):
        pltpu.sync_copy(u_sh.at[pl.ds(rstart + i * (ZROWS // 8), ZROWS // 8)],
                        zbuf)
        pltpu.sync_copy(zbuf,
                        u_out.at[c, pl.ds(rstart + i * (ZROWS // 8),
                                          ZROWS // 8)])
    pltpu.sync_copy(den_sh.at[pl.ds(rstart, ZROWS)], dzb)
    pltpu.sync_copy(dzb, den_out.at[pl.ds(c * N + rstart, ZROWS)])


_sc_edge = functools.partial(
    pl.kernel,
    out_type=(
        jax.ShapeDtypeStruct((NUM_SC, N, D), jnp.float32),
        jax.ShapeDtypeStruct((NUM_SC * N,), jnp.float32),
    ),
    mesh=plsc.VectorSubcoreMesh(core_axis_name="c", subcore_axis_name="s"),
    compiler_params=pltpu.CompilerParams(needs_layout_passes=False),
    scratch_types=[
        pltpu.VMEM((N,), jnp.float32),
        pltpu.VMEM((N,), jnp.float32),
        pltpu.VMEM((CHUNK,), jnp.int32),
        pltpu.VMEM((CHUNK,), jnp.int32),
        pltpu.VMEM((CHUNK,), jnp.float32),
        pltpu.VMEM((CHUNK, D), jnp.float32),
        pltpu.VMEM((ZROWS // 8, D), jnp.float32),
        pltpu.VMEM((ZROWS,), jnp.float32),
        pltpu.VMEM_SHARED((N, D), jnp.float32),
        pltpu.VMEM_SHARED((N,), jnp.float32),
        pltpu.SemaphoreType.DMA,
    ],
)(_sc_edge_body)


def kernel(x, edge_index, W, a_src, a_dst, bias):
    a2 = jnp.concatenate(
        [a_src[None, :], a_dst[None, :], jnp.zeros((6, D), jnp.float32)], 0)
    h, sa = _matmul(x, W, a2)
    as_ = sa[:, 0]
    ad_ = sa[:, 1]
    u, dflat = _sc_edge(edge_index[0], edge_index[1], as_, ad_, h)
    vecs = jnp.stack([as_, ad_, dflat[:N], dflat[N:]], axis=1)
    return _combine(u, h, vecs, bias.reshape(1, D))


# speedup vs baseline: 32.4368x; 1.2181x over previous
import functools

import jax
import jax.numpy as jnp
from jax import lax
from jax.experimental import pallas as pl
from jax.experimental.pallas import tpu as pltpu
from jax.experimental.pallas import tpu_sc as plsc

N = 10000
E = 320000
D = 128

NUM_SC = 2
NUM_TILES = 16
NUM_WORKERS = NUM_SC * NUM_TILES
EPW = E // NUM_WORKERS
CHUNK = 80
NCHUNK = EPW // CHUNK
ZROWS = 640
ZSTRIDE = 632
BM = 2000


def _mm_body(x_ref, w_ref, a2_ref, h_ref, sat_ref):
    h = jnp.dot(x_ref[...], w_ref[...], preferred_element_type=jnp.float32)
    h_ref[...] = h
    sat_ref[...] = lax.dot_general(
        h, a2_ref[...], (((1,), (1,)), ((), ())),
        preferred_element_type=jnp.float32)


def _matmul(x, W, a2):
    return pl.pallas_call(
        _mm_body,
        grid=(N // BM,),
        in_specs=[
            pl.BlockSpec((BM, D), lambda i: (i, 0)),
            pl.BlockSpec((D, D), lambda i: (0, 0)),
            pl.BlockSpec((8, D), lambda i: (0, 0)),
        ],
        out_specs=[
            pl.BlockSpec((BM, D), lambda i: (i, 0)),
            pl.BlockSpec((BM, 8), lambda i: (i, 0)),
        ],
        out_shape=[
            jax.ShapeDtypeStruct((N, D), jnp.float32),
            jax.ShapeDtypeStruct((N, 8), jnp.float32),
        ],
    )(x, W, a2)


def _comb_body(u_ref, h_ref, v_ref, b_ref, o_ref):
    v = v_ref[...]
    ws = v[:, 0:1] + v[:, 1:2]
    ws = jnp.exp(jnp.where(ws >= 0.0, ws, 0.2 * ws))
    den = v[:, 2:3] + v[:, 3:4] + ws + 1e-16
    o_ref[...] = (u_ref[0] + u_ref[1] + ws * h_ref[...]) / den + b_ref[...]


def _combine(u, h, vecs, bias2d):
    return pl.pallas_call(
        _comb_body,
        grid=(N // BM,),
        in_specs=[
            pl.BlockSpec((2, BM, D), lambda i: (0, i, 0)),
            pl.BlockSpec((BM, D), lambda i: (i, 0)),
            pl.BlockSpec((BM, 4), lambda i: (i, 0)),
            pl.BlockSpec((1, D), lambda i: (0, 0)),
        ],
        out_specs=pl.BlockSpec((BM, D), lambda i: (i, 0)),
        out_shape=jax.ShapeDtypeStruct((N, D), jnp.float32),
    )(u, h, vecs, bias2d)


def _sc_edge_body(src_hbm, dst_hbm, as_hbm, ad_hbm, h_hbm,
                  u_out, den_out,
                  as_l, ad_l, src_b, dst_b, w_b, rows_b, u_sh,
                  den_sh, gsem, ssem):
    c = lax.axis_index("c")
    s = lax.axis_index("s")
    wid = s * NUM_SC + c

    pltpu.sync_copy(as_hbm, as_l)
    pltpu.sync_copy(ad_hbm, ad_l)

    z16 = jnp.zeros((16,), jnp.float32)
    for r in range(CHUNK):
        for k in range(D // 16):
            rows_b[0, r, pl.ds(k * 16, 16)] = z16
    rstart = jnp.minimum(s * ZSTRIDE, N - ZROWS)
    for i in range(8):
        pltpu.sync_copy(rows_b.at[0], u_sh.at[pl.ds(rstart + i * CHUNK,
                                                    CHUNK)])
    for i in range(ZROWS // D):
        pltpu.sync_copy(rows_b.at[0, 0],
                        den_sh.at[pl.ds(rstart + i * D, D)])
    plsc.subcore_barrier()

    ebase = wid * EPW

    def fetch(j, b):
        off = ebase + j * CHUNK
        pltpu.sync_copy(src_hbm.at[pl.ds(off, CHUNK)], src_b.at[b])
        pltpu.sync_copy(dst_hbm.at[pl.ds(off, CHUNK)], dst_b.at[b])
        pltpu.async_copy(h_hbm.at[src_b.at[b]], rows_b.at[b], gsem.at[b])

    def wait_scatter(b):
        pltpu.make_async_copy(rows_b.at[b], u_sh.at[dst_b.at[b]],
                              ssem.at[b]).wait()

    fetch(0, 0)

    def outer(jj, carry):
        for b in range(2):
            ob = 1 - b
            j = jj * 2 + b

            @pl.when(jnp.logical_and(j >= 1, j < NCHUNK - 1))
            def _drain():
                wait_scatter(ob)

            @pl.when(j < NCHUNK - 1)
            def _prefetch():
                fetch(j + 1, ob)

            @pl.when(j <= NCHUNK - 1)
            def _process():
                for g in range(CHUNK // 16):
                    si = src_b[b, pl.ds(g * 16, 16)]
                    di = dst_b[b, pl.ds(g * 16, 16)]
                    e = (plsc.load_gather(as_l, [si])
                         + plsc.load_gather(ad_l, [di]))
                    e = jnp.where(e >= 0.0, e, e * 0.2)
                    w_b[b, pl.ds(g * 16, 16)] = jnp.exp(e)
                pltpu.make_async_copy(h_hbm.at[src_b.at[b]], rows_b.at[b],
                                      gsem.at[b]).wait()
                for g in range(CHUNK // 16):
                    wv = w_b[b, pl.ds(g * 16, 16)]
                    for l in range(16):
                        ws = jnp.full((16,), wv[l], jnp.float32)
                        r = g * 16 + l
                        for k in range(D // 16):
                            rows_b[b, r, pl.ds(k * 16, 16)] = (
                                rows_b[b, r, pl.ds(k * 16, 16)] * ws)
                pltpu.sync_copy(w_b.at[b], den_sh.at[dst_b.at[b]], add=True)
                pltpu.async_copy(rows_b.at[b], u_sh.at[dst_b.at[b]],
                                 ssem.at[b], add=True)
        return carry

    lax.fori_loop(0, (NCHUNK + 1) // 2, outer, 0)
    wait_scatter(0)
    wait_scatter(1)
    plsc.subcore_barrier()

    for i in range(8):
        pltpu.sync_copy(u_sh.at[pl.ds(rstart + i * CHUNK, CHUNK)],
                        rows_b.at[0])
        pltpu.sync_copy(rows_b.at[0],
                        u_out.at[c, pl.ds(rstart + i * CHUNK, CHUNK)])
    for i in range(ZROWS // D):
        pltpu.sync_copy(den_sh.at[pl.ds(rstart + i * D, D)],
                        rows_b.at[0, 0])
        pltpu.sync_copy(rows_b.at[0, 0],
                        den_out.at[pl.ds(c * N + rstart + i * D, D)])


_sc_edge = functools.partial(
    pl.kernel,
    out_type=(
        jax.ShapeDtypeStruct((NUM_SC, N, D), jnp.float32),
        jax.ShapeDtypeStruct((NUM_SC * N,), jnp.float32),
    ),
    mesh=plsc.VectorSubcoreMesh(core_axis_name="c", subcore_axis_name="s"),
    compiler_params=pltpu.CompilerParams(needs_layout_passes=False),
    scratch_types=[
        pltpu.VMEM((N,), jnp.float32),
        pltpu.VMEM((N,), jnp.float32),
        pltpu.VMEM((2, CHUNK), jnp.int32),
        pltpu.VMEM((2, CHUNK), jnp.int32),
        pltpu.VMEM((2, CHUNK), jnp.float32),
        pltpu.VMEM((2, CHUNK, D), jnp.float32),
        pltpu.VMEM_SHARED((N, D), jnp.float32),
        pltpu.VMEM_SHARED((N,), jnp.float32),
        pltpu.SemaphoreType.DMA((2,)),
        pltpu.SemaphoreType.DMA((2,)),
    ],
)(_sc_edge_body)


def kernel(x, edge_index, W, a_src, a_dst, bias):
    a2 = jnp.concatenate(
        [a_src[None, :], a_dst[None, :], jnp.zeros((6, D), jnp.float32)], 0)
    h, sa = _matmul(x, W, a2)
    as_ = sa[:, 0]
    ad_ = sa[:, 1]
    u, dflat = _sc_edge(edge_index[0], edge_index[1], as_, ad_, h)
    vecs = jnp.stack([as_, ad_, dflat[:N], dflat[N:]], axis=1)
    return _combine(u, h, vecs, bias.reshape(1, D))


# speedup vs baseline: 32.8738x; 1.0135x over previous
import functools

import jax
import jax.numpy as jnp
from jax import lax
from jax.experimental import pallas as pl
from jax.experimental.pallas import tpu as pltpu
from jax.experimental.pallas import tpu_sc as plsc

N = 10000
E = 320000
D = 128

NUM_SC = 2
NUM_TILES = 16
NUM_WORKERS = NUM_SC * NUM_TILES
EPW = E // NUM_WORKERS
CHUNK = 80
NCHUNK = EPW // CHUNK
ZROWS = 640
ZSTRIDE = 632
BM = 2000


def _mm_body(x_ref, w_ref, a2_ref, h_ref, sat_ref):
    h = jnp.dot(x_ref[...], w_ref[...], preferred_element_type=jnp.float32)
    h_ref[...] = h
    sat_ref[...] = lax.dot_general(
        h, a2_ref[...], (((1,), (1,)), ((), ())),
        preferred_element_type=jnp.float32)


def _matmul(x, W, a2):
    return pl.pallas_call(
        _mm_body,
        grid=(N // BM,),
        in_specs=[
            pl.BlockSpec((BM, D), lambda i: (i, 0)),
            pl.BlockSpec((D, D), lambda i: (0, 0)),
            pl.BlockSpec((8, D), lambda i: (0, 0)),
        ],
        out_specs=[
            pl.BlockSpec((BM, D), lambda i: (i, 0)),
            pl.BlockSpec((BM, 8), lambda i: (i, 0)),
        ],
        out_shape=[
            jax.ShapeDtypeStruct((N, D), jnp.float32),
            jax.ShapeDtypeStruct((N, 8), jnp.float32),
        ],
    )(x, W, a2)


def _comb_body(u_ref, h_ref, v_ref, b_ref, o_ref):
    v = v_ref[...]
    ws = v[:, 0:1] + v[:, 1:2]
    ws = jnp.exp(jnp.where(ws >= 0.0, ws, 0.2 * ws))
    den = v[:, 2:3] + v[:, 3:4] + ws + 1e-16
    o_ref[...] = (u_ref[0] + u_ref[1] + ws * h_ref[...]) / den + b_ref[...]


def _combine(u, h, vecs, bias2d):
    return pl.pallas_call(
        _comb_body,
        grid=(N // BM,),
        in_specs=[
            pl.BlockSpec((2, BM, D), lambda i: (0, i, 0)),
            pl.BlockSpec((BM, D), lambda i: (i, 0)),
            pl.BlockSpec((BM, 4), lambda i: (i, 0)),
            pl.BlockSpec((1, D), lambda i: (0, 0)),
        ],
        out_specs=pl.BlockSpec((BM, D), lambda i: (i, 0)),
        out_shape=jax.ShapeDtypeStruct((N, D), jnp.float32),
    )(u, h, vecs, bias2d)


def _sc_edge_body(src_hbm, dst_hbm, as_hbm, ad_hbm, h_hbm,
                  u_out, den_out,
                  as_l, ad_l, src_b, dst_b, w_b, rows_b, u_sh,
                  den_sh, gsem, ssem, dsem):
    c = lax.axis_index("c")
    s = lax.axis_index("s")
    wid = s * NUM_SC + c

    pltpu.sync_copy(as_hbm, as_l)
    pltpu.sync_copy(ad_hbm, ad_l)

    z16 = jnp.zeros((16,), jnp.float32)
    for r in range(CHUNK):
        for k in range(D // 16):
            rows_b[0, r, pl.ds(k * 16, 16)] = z16
    rstart = jnp.minimum(s * ZSTRIDE, N - ZROWS)
    for i in range(8):
        pltpu.sync_copy(rows_b.at[0], u_sh.at[pl.ds(rstart + i * CHUNK,
                                                    CHUNK)])
    for i in range(ZROWS // D):
        pltpu.sync_copy(rows_b.at[0, 0],
                        den_sh.at[pl.ds(rstart + i * D, D)])
    plsc.subcore_barrier()

    ebase = wid * EPW

    def fetch(j, b):
        off = ebase + j * CHUNK
        pltpu.sync_copy(src_hbm.at[pl.ds(off, CHUNK)], src_b.at[b])
        pltpu.sync_copy(dst_hbm.at[pl.ds(off, CHUNK)], dst_b.at[b])
        pltpu.async_copy(h_hbm.at[src_b.at[b]], rows_b.at[b], gsem.at[b])

    def wait_scatter(b):
        pltpu.make_async_copy(rows_b.at[b], u_sh.at[dst_b.at[b]],
                              ssem.at[b]).wait()
        pltpu.make_async_copy(w_b.at[b], den_sh.at[dst_b.at[b]],
                              dsem.at[b]).wait()

    fetch(0, 0)

    def outer(jj, carry):
        for b in range(2):
            ob = 1 - b
            j = jj * 2 + b

            @pl.when(jnp.logical_and(j >= 1, j < NCHUNK - 1))
            def _drain():
                wait_scatter(ob)

            @pl.when(j < NCHUNK - 1)
            def _prefetch():
                fetch(j + 1, ob)

            @pl.when(j <= NCHUNK - 1)
            def _process():
                for g in range(CHUNK // 16):
                    si = src_b[b, pl.ds(g * 16, 16)]
                    di = dst_b[b, pl.ds(g * 16, 16)]
                    e = (plsc.load_gather(as_l, [si])
                         + plsc.load_gather(ad_l, [di]))
                    e = jnp.where(e >= 0.0, e, e * 0.2)
                    w_b[b, pl.ds(g * 16, 16)] = jnp.exp(e)
                pltpu.make_async_copy(h_hbm.at[src_b.at[b]], rows_b.at[b],
                                      gsem.at[b]).wait()
                for g in range(CHUNK // 16):
                    wv = w_b[b, pl.ds(g * 16, 16)]
                    for l in range(16):
                        ws = jnp.full((16,), wv[l], jnp.float32)
                        r = g * 16 + l
                        for k in range(D // 16):
                            rows_b[b, r, pl.ds(k * 16, 16)] = (
                                rows_b[b, r, pl.ds(k * 16, 16)] * ws)
                pltpu.async_copy(w_b.at[b], den_sh.at[dst_b.at[b]],
                                 dsem.at[b], add=True)
                pltpu.async_copy(rows_b.at[b], u_sh.at[dst_b.at[b]],
                                 ssem.at[b], add=True)
        return carry

    lax.fori_loop(0, (NCHUNK + 1) // 2, outer, 0)
    wait_scatter(0)
    wait_scatter(1)
    plsc.subcore_barrier()

    for i in range(8):
        pltpu.sync_copy(u_sh.at[pl.ds(rstart + i * CHUNK, CHUNK)],
                        rows_b.at[0])
        pltpu.sync_copy(rows_b.at[0],
                        u_out.at[c, pl.ds(rstart + i * CHUNK, CHUNK)])
    for i in range(ZROWS // D):
        pltpu.sync_copy(den_sh.at[pl.ds(rstart + i * D, D)],
                        rows_b.at[0, 0])
        pltpu.sync_copy(rows_b.at[0, 0],
                        den_out.at[pl.ds(c * N + rstart + i * D, D)])


_sc_edge = functools.partial(
    pl.kernel,
    out_type=(
        jax.ShapeDtypeStruct((NUM_SC, N, D), jnp.float32),
        jax.ShapeDtypeStruct((NUM_SC * N,), jnp.float32),
    ),
    mesh=plsc.VectorSubcoreMesh(core_axis_name="c", subcore_axis_name="s"),
    compiler_params=pltpu.CompilerParams(needs_layout_passes=False),
    scratch_types=[
        pltpu.VMEM((N,), jnp.float32),
        pltpu.VMEM((N,), jnp.float32),
        pltpu.VMEM((2, CHUNK), jnp.int32),
        pltpu.VMEM((2, CHUNK), jnp.int32),
        pltpu.VMEM((2, CHUNK), jnp.float32),
        pltpu.VMEM((2, CHUNK, D), jnp.float32),
        pltpu.VMEM_SHARED((N, D), jnp.float32),
        pltpu.VMEM_SHARED((N,), jnp.float32),
        pltpu.SemaphoreType.DMA((2,)),
        pltpu.SemaphoreType.DMA((2,)),
        pltpu.SemaphoreType.DMA((2,)),
    ],
)(_sc_edge_body)


def kernel(x, edge_index, W, a_src, a_dst, bias):
    a2 = jnp.concatenate(
        [a_src[None, :], a_dst[None, :], jnp.zeros((6, D), jnp.float32)], 0)
    h, sa = _matmul(x, W, a2)
    as_ = sa[:, 0]
    ad_ = sa[:, 1]
    u, dflat = _sc_edge(edge_index[0], edge_index[1], as_, ad_, h)
    vecs = jnp.stack([as_, ad_, dflat[:N], dflat[N:]], axis=1)
    return _combine(u, h, vecs, bias.reshape(1, D))


# speedup vs baseline: 42.4265x; 1.2906x over previous
import functools

import jax
import jax.numpy as jnp
from jax import lax
from jax.experimental import pallas as pl
from jax.experimental.pallas import tpu as pltpu
from jax.experimental.pallas import tpu_sc as plsc

N = 10000
E = 320000
D = 128

NUM_SC = 2
NUM_TILES = 16
NUM_WORKERS = NUM_SC * NUM_TILES
EPW = E // NUM_WORKERS
CHUNK = 80
NCHUNK = EPW // CHUNK
ZROWS = 640
ZSTRIDE = 632
BM = 2000


def _mm_body(x_ref, w_ref, a2_ref, h_ref, sat_ref):
    h = jnp.dot(x_ref[...], w_ref[...], preferred_element_type=jnp.float32)
    h_ref[...] = h
    sat_ref[...] = lax.dot_general(
        h, a2_ref[...], (((1,), (1,)), ((), ())),
        preferred_element_type=jnp.float32)


def _matmul(x, W, a2):
    return pl.pallas_call(
        _mm_body,
        grid=(N // BM,),
        in_specs=[
            pl.BlockSpec((BM, D), lambda i: (i, 0)),
            pl.BlockSpec((D, D), lambda i: (0, 0)),
            pl.BlockSpec((8, D), lambda i: (0, 0)),
        ],
        out_specs=[
            pl.BlockSpec((BM, D), lambda i: (i, 0)),
            pl.BlockSpec((BM, 8), lambda i: (i, 0)),
        ],
        out_shape=[
            jax.ShapeDtypeStruct((N, D), jnp.float32),
            jax.ShapeDtypeStruct((N, 8), jnp.float32),
        ],
    )(x, W, a2)


def _comb_body(u_ref, h_ref, v_ref, b_ref, o_ref):
    v = v_ref[...]
    ws = v[:, 0:1] + v[:, 1:2]
    ws = jnp.exp(jnp.where(ws >= 0.0, ws, 0.2 * ws))
    den = v[:, 2:3] + v[:, 3:4] + ws + 1e-16
    o_ref[...] = (u_ref[0] + u_ref[1] + ws * h_ref[...]) / den + b_ref[...]


def _combine(u, h, vecs, bias2d):
    return pl.pallas_call(
        _comb_body,
        grid=(N // BM,),
        in_specs=[
            pl.BlockSpec((2, BM, D), lambda i: (0, i, 0)),
            pl.BlockSpec((BM, D), lambda i: (i, 0)),
            pl.BlockSpec((BM, 4), lambda i: (i, 0)),
            pl.BlockSpec((1, D), lambda i: (0, 0)),
        ],
        out_specs=pl.BlockSpec((BM, D), lambda i: (i, 0)),
        out_shape=jax.ShapeDtypeStruct((N, D), jnp.float32),
    )(u, h, vecs, bias2d)


def _sc_edge_body(ei_hbm, as_hbm, ad_hbm, h_hbm,
                  u_out, den_out,
                  sd_b, asg_b, adg_b, w_b, rows_b, u_sh,
                  den_sh, isem, lsem, gsem, ssem, dsem):
    c = lax.axis_index("c")
    s = lax.axis_index("s")
    wid = s * NUM_SC + c

    z16 = jnp.zeros((16,), jnp.float32)
    for r in range(CHUNK):
        for k in range(D // 16):
            rows_b[0, r, pl.ds(k * 16, 16)] = z16
    rstart = jnp.minimum(s * ZSTRIDE, N - ZROWS)
    for i in range(8):
        pltpu.sync_copy(rows_b.at[0], u_sh.at[pl.ds(rstart + i * CHUNK,
                                                    CHUNK)])
    for i in range(ZROWS // D):
        pltpu.sync_copy(rows_b.at[0, 0],
                        den_sh.at[pl.ds(rstart + i * D, D)])
    plsc.subcore_barrier()

    rbase = wid * NCHUNK

    def issue_idx(j, b):
        pltpu.async_copy(ei_hbm.at[rbase + j], sd_b.at[b], isem.at[b])

    def wait_idx(b):
        pltpu.make_async_copy(ei_hbm.at[rbase], sd_b.at[b], isem.at[b]).wait()

    def issue_gathers(b):
        pltpu.async_copy(h_hbm.at[sd_b.at[b, 0]], rows_b.at[b], gsem.at[b])
        pltpu.async_copy(as_hbm.at[sd_b.at[b, 0]], asg_b.at[b], lsem.at[b])
        pltpu.async_copy(ad_hbm.at[sd_b.at[b, 1]], adg_b.at[b], lsem.at[b])

    def wait_gathers(b):
        pltpu.make_async_copy(h_hbm.at[sd_b.at[b, 0]], rows_b.at[b],
                              gsem.at[b]).wait()
        pltpu.make_async_copy(as_hbm.at[sd_b.at[b, 0]], asg_b.at[b],
                              lsem.at[b]).wait()
        pltpu.make_async_copy(ad_hbm.at[sd_b.at[b, 1]], adg_b.at[b],
                              lsem.at[b]).wait()

    def wait_scatter(b):
        pltpu.make_async_copy(rows_b.at[b], u_sh.at[sd_b.at[b, 1]],
                              ssem.at[b]).wait()
        pltpu.make_async_copy(w_b.at[b], den_sh.at[sd_b.at[b, 1]],
                              dsem.at[b]).wait()

    issue_idx(0, 0)
    issue_idx(1, 1)
    wait_idx(0)
    issue_gathers(0)

    def outer(jj, carry):
        for b in range(3):
            j = jj * 3 + b
            s_idx = (b + 2) % 3
            s_g = (b + 1) % 3

            @pl.when(jnp.logical_and(j >= 1, j <= NCHUNK - 3))
            def _drain():
                wait_scatter(s_idx)

            @pl.when(j <= NCHUNK - 3)
            def _idx():
                issue_idx(j + 2, s_idx)

            @pl.when(j <= NCHUNK - 2)
            def _gath():
                wait_idx(s_g)
                issue_gathers(s_g)

            @pl.when(j <= NCHUNK - 1)
            def _process():
                wait_gathers(b)
                for g in range(CHUNK // 16):
                    e = (asg_b[b, pl.ds(g * 16, 16)]
                         + adg_b[b, pl.ds(g * 16, 16)])
                    e = jnp.where(e >= 0.0, e, e * 0.2)
                    w_b[b, pl.ds(g * 16, 16)] = jnp.exp(e)
                for g in range(CHUNK // 16):
                    wv = w_b[b, pl.ds(g * 16, 16)]
                    for l in range(16):
                        ws = jnp.full((16,), wv[l], jnp.float32)
                        r = g * 16 + l
                        for k in range(D // 16):
                            rows_b[b, r, pl.ds(k * 16, 16)] = (
                                rows_b[b, r, pl.ds(k * 16, 16)] * ws)
                pltpu.async_copy(w_b.at[b], den_sh.at[sd_b.at[b, 1]],
                                 dsem.at[b], add=True)
                pltpu.async_copy(rows_b.at[b], u_sh.at[sd_b.at[b, 1]],
                                 ssem.at[b], add=True)
        return carry

    lax.fori_loop(0, (NCHUNK + 2) // 3, outer, 0)
    wait_scatter(0)
    wait_scatter(1)
    wait_scatter(2)
    plsc.subcore_barrier()

    for i in range(8):
        pltpu.sync_copy(u_sh.at[pl.ds(rstart + i * CHUNK, CHUNK)],
                        rows_b.at[0])
        pltpu.sync_copy(rows_b.at[0],
                        u_out.at[c, pl.ds(rstart + i * CHUNK, CHUNK)])
    for i in range(ZROWS // D):
        pltpu.sync_copy(den_sh.at[pl.ds(rstart + i * D, D)],
                        rows_b.at[0, 0])
        pltpu.sync_copy(rows_b.at[0, 0],
                        den_out.at[pl.ds(c * N + rstart + i * D, D)])


_sc_edge = functools.partial(
    pl.kernel,
    out_type=(
        jax.ShapeDtypeStruct((NUM_SC, N, D), jnp.float32),
        jax.ShapeDtypeStruct((NUM_SC * N,), jnp.float32),
    ),
    mesh=plsc.VectorSubcoreMesh(core_axis_name="c", subcore_axis_name="s"),
    compiler_params=pltpu.CompilerParams(needs_layout_passes=False),
    scratch_types=[
        pltpu.VMEM((3, 2, CHUNK), jnp.int32),
        pltpu.VMEM((3, CHUNK), jnp.float32),
        pltpu.VMEM((3, CHUNK), jnp.float32),
        pltpu.VMEM((3, CHUNK), jnp.float32),
        pltpu.VMEM((3, CHUNK, D), jnp.float32),
        pltpu.VMEM_SHARED((N, D), jnp.float32),
        pltpu.VMEM_SHARED((N,), jnp.float32),
        pltpu.SemaphoreType.DMA((3,)),
        pltpu.SemaphoreType.DMA((3,)),
        pltpu.SemaphoreType.DMA((3,)),
        pltpu.SemaphoreType.DMA((3,)),
        pltpu.SemaphoreType.DMA((3,)),
    ],
)(_sc_edge_body)


def kernel(x, edge_index, W, a_src, a_dst, bias):
    a2 = jnp.concatenate(
        [a_src[None, :], a_dst[None, :], jnp.zeros((6, D), jnp.float32)], 0)
    h, sa = _matmul(x, W, a2)
    as_ = sa[:, 0]
    ad_ = sa[:, 1]
    ei3 = edge_index.reshape(2, E // CHUNK, CHUNK).transpose(1, 0, 2)
    u, dflat = _sc_edge(ei3, as_, ad_, h)
    vecs = jnp.stack([as_, ad_, dflat[:N], dflat[N:]], axis=1)
    return _combine(u, h, vecs, bias.reshape(1, D))


# speedup vs baseline: 45.7386x; 1.0781x over previous
import functools

import jax
import jax.numpy as jnp
from jax import lax
from jax.experimental import pallas as pl
from jax.experimental.pallas import tpu as pltpu
from jax.experimental.pallas import tpu_sc as plsc

N = 10000
E = 320000
D = 128

NUM_SC = 2
NUM_TILES = 16
NUM_WORKERS = NUM_SC * NUM_TILES
EPW = E // NUM_WORKERS
CHUNK = 80
NCHUNK = EPW // CHUNK
NSLOT = 4
ZROWS = 640
ZSTRIDE = 632
BM = 2000


def _mm_body(x_ref, w_ref, a2_ref, h_ref, sat_ref):
    h = jnp.dot(x_ref[...], w_ref[...], preferred_element_type=jnp.float32)
    h_ref[...] = h
    sat_ref[...] = lax.dot_general(
        h, a2_ref[...], (((1,), (1,)), ((), ())),
        preferred_element_type=jnp.float32)


def _matmul(x, W, a2):
    return pl.pallas_call(
        _mm_body,
        grid=(N // BM,),
        in_specs=[
            pl.BlockSpec((BM, D), lambda i: (i, 0)),
            pl.BlockSpec((D, D), lambda i: (0, 0)),
            pl.BlockSpec((8, D), lambda i: (0, 0)),
        ],
        out_specs=[
            pl.BlockSpec((BM, D), lambda i: (i, 0)),
            pl.BlockSpec((BM, 8), lambda i: (i, 0)),
        ],
        out_shape=[
            jax.ShapeDtypeStruct((N, D), jnp.float32),
            jax.ShapeDtypeStruct((N, 8), jnp.float32),
        ],
    )(x, W, a2)


def _comb_body(u_ref, h_ref, v_ref, b_ref, o_ref):
    v = v_ref[...]
    ws = v[:, 0:1] + v[:, 1:2]
    ws = jnp.exp(jnp.where(ws >= 0.0, ws, 0.2 * ws))
    den = v[:, 2:3] + v[:, 3:4] + ws + 1e-16
    o_ref[...] = (u_ref[0] + u_ref[1] + ws * h_ref[...]) / den + b_ref[...]


def _combine(u, h, vecs, bias2d):
    return pl.pallas_call(
        _comb_body,
        grid=(N // BM,),
        in_specs=[
            pl.BlockSpec((2, BM, D), lambda i: (0, i, 0)),
            pl.BlockSpec((BM, D), lambda i: (i, 0)),
            pl.BlockSpec((BM, 4), lambda i: (i, 0)),
            pl.BlockSpec((1, D), lambda i: (0, 0)),
        ],
        out_specs=pl.BlockSpec((BM, D), lambda i: (i, 0)),
        out_shape=jax.ShapeDtypeStruct((N, D), jnp.float32),
    )(u, h, vecs, bias2d)


def _sc_edge_body(ei_hbm, as_hbm, ad_hbm, h_hbm,
                  u_out, den_out,
                  sd_b, asg_b, adg_b, w_b, rows_b, u_sh,
                  den_sh, isem, lsem, gsem, ssem, dsem):
    c = lax.axis_index("c")
    s = lax.axis_index("s")
    wid = s * NUM_SC + c

    z16 = jnp.zeros((16,), jnp.float32)
    for r in range(CHUNK):
        for k in range(D // 16):
            rows_b[0, r, pl.ds(k * 16, 16)] = z16
    rstart = jnp.minimum(s * ZSTRIDE, N - ZROWS)
    for i in range(8):
        pltpu.sync_copy(rows_b.at[0], u_sh.at[pl.ds(rstart + i * CHUNK,
                                                    CHUNK)])
    for i in range(ZROWS // D):
        pltpu.sync_copy(rows_b.at[0, 0],
                        den_sh.at[pl.ds(rstart + i * D, D)])
    plsc.subcore_barrier()

    rbase = wid * NCHUNK

    def issue_idx(j, b):
        pltpu.async_copy(ei_hbm.at[rbase + j], sd_b.at[b], isem.at[b])

    def wait_idx(b):
        pltpu.make_async_copy(ei_hbm.at[rbase], sd_b.at[b], isem.at[b]).wait()

    def issue_gathers(b):
        pltpu.async_copy(h_hbm.at[sd_b.at[b, 0]], rows_b.at[b], gsem.at[b])
        pltpu.async_copy(as_hbm.at[sd_b.at[b, 0]], asg_b.at[b], lsem.at[b])
        pltpu.async_copy(ad_hbm.at[sd_b.at[b, 1]], adg_b.at[b], lsem.at[b])

    def wait_gathers(b):
        pltpu.make_async_copy(h_hbm.at[sd_b.at[b, 0]], rows_b.at[b],
                              gsem.at[b]).wait()
        pltpu.make_async_copy(as_hbm.at[sd_b.at[b, 0]], asg_b.at[b],
                              lsem.at[b]).wait()
        pltpu.make_async_copy(ad_hbm.at[sd_b.at[b, 1]], adg_b.at[b],
                              lsem.at[b]).wait()

    def wait_scatter(b):
        pltpu.make_async_copy(rows_b.at[b], u_sh.at[sd_b.at[b, 1]],
                              ssem.at[b]).wait()
        pltpu.make_async_copy(w_b.at[b], den_sh.at[sd_b.at[b, 1]],
                              dsem.at[b]).wait()

    issue_idx(0, 0)
    issue_idx(1, 1)
    wait_idx(0)
    issue_gathers(0)

    def outer(jj, carry):
        for b in range(NSLOT):
            j = jj * NSLOT + b
            s_idx = (b + 2) % NSLOT
            s_g = (b + 1) % NSLOT

            @pl.when(jnp.logical_and(j >= 2, j <= NCHUNK - 3))
            def _drain():
                wait_scatter(s_idx)

            @pl.when(j <= NCHUNK - 3)
            def _idx():
                issue_idx(j + 2, s_idx)

            @pl.when(j <= NCHUNK - 2)
            def _gath():
                wait_idx(s_g)
                issue_gathers(s_g)

            @pl.when(j <= NCHUNK - 1)
            def _process():
                wait_gathers(b)
                for g in range(CHUNK // 16):
                    e = (asg_b[b, pl.ds(g * 16, 16)]
                         + adg_b[b, pl.ds(g * 16, 16)])
                    e = jnp.where(e >= 0.0, e, e * 0.2)
                    w_b[b, pl.ds(g * 16, 16)] = jnp.exp(e)
                for g in range(CHUNK // 16):
                    wv = w_b[b, pl.ds(g * 16, 16)]
                    for l in range(16):
                        ws = jnp.full((16,), wv[l], jnp.float32)
                        r = g * 16 + l
                        for k in range(D // 16):
                            rows_b[b, r, pl.ds(k * 16, 16)] = (
                                rows_b[b, r, pl.ds(k * 16, 16)] * ws)
                pltpu.async_copy(w_b.at[b], den_sh.at[sd_b.at[b, 1]],
                                 dsem.at[b], add=True)
                pltpu.async_copy(rows_b.at[b], u_sh.at[sd_b.at[b, 1]],
                                 ssem.at[b], add=True)
        return carry

    lax.fori_loop(0, (NCHUNK + NSLOT - 1) // NSLOT, outer, 0)
    for b in range(NSLOT):
        wait_scatter(b)
    plsc.subcore_barrier()

    for i in range(8):
        pltpu.sync_copy(u_sh.at[pl.ds(rstart + i * CHUNK, CHUNK)],
                        rows_b.at[0])
        pltpu.sync_copy(rows_b.at[0],
                        u_out.at[c, pl.ds(rstart + i * CHUNK, CHUNK)])
    for i in range(ZROWS // D):
        pltpu.sync_copy(den_sh.at[pl.ds(rstart + i * D, D)],
                        rows_b.at[0, 0])
        pltpu.sync_copy(rows_b.at[0, 0],
                        den_out.at[pl.ds(c * N + rstart + i * D, D)])


_sc_edge = functools.partial(
    pl.kernel,
    out_type=(
        jax.ShapeDtypeStruct((NUM_SC, N, D), jnp.float32),
        jax.ShapeDtypeStruct((NUM_SC * N,), jnp.float32),
    ),
    mesh=plsc.VectorSubcoreMesh(core_axis_name="c", subcore_axis_name="s"),
    compiler_params=pltpu.CompilerParams(needs_layout_passes=False),
    scratch_types=[
        pltpu.VMEM((NSLOT, 2, CHUNK), jnp.int32),
        pltpu.VMEM((NSLOT, CHUNK), jnp.float32),
        pltpu.VMEM((NSLOT, CHUNK), jnp.float32),
        pltpu.VMEM((NSLOT, CHUNK), jnp.float32),
        pltpu.VMEM((NSLOT, CHUNK, D), jnp.float32),
        pltpu.VMEM_SHARED((N, D), jnp.float32),
        pltpu.VMEM_SHARED((N,), jnp.float32),
        pltpu.SemaphoreType.DMA((NSLOT,)),
        pltpu.SemaphoreType.DMA((NSLOT,)),
        pltpu.SemaphoreType.DMA((NSLOT,)),
        pltpu.SemaphoreType.DMA((NSLOT,)),
        pltpu.SemaphoreType.DMA((NSLOT,)),
    ],
)(_sc_edge_body)


def kernel(x, edge_index, W, a_src, a_dst, bias):
    a2 = jnp.concatenate(
        [a_src[None, :], a_dst[None, :], jnp.zeros((6, D), jnp.float32)], 0)
    h, sa = _matmul(x, W, a2)
    as_ = sa[:, 0]
    ad_ = sa[:, 1]
    ei3 = edge_index.reshape(2, E // CHUNK, CHUNK).transpose(1, 0, 2)
    u, dflat = _sc_edge(ei3, as_, ad_, h)
    vecs = jnp.stack([as_, ad_, dflat[:N], dflat[N:]], axis=1)
    return _combine(u, h, vecs, bias.reshape(1, D))
